# baseline (speedup 1.0000x reference)
import functools

import jax
import jax.numpy as jnp
from jax import lax
from jax.experimental import pallas as pl
from jax.experimental.pallas import tpu as pltpu
from jax.experimental.pallas import tpu_sc as plsc

_EMB = 64
_BATCH = 16384
_NC = 2
_NS = 16
_NW = _NC * _NS
_BPW = _BATCH // _NW
_CHUNK = 128
_NCH = _BPW // _CHUNK
_GROUPS = _BPW // 16


def _body(tw, cw, te, ce, out, idx_t, idx_c, rows_t, rows_c, accv, sem):
    wid = lax.axis_index("s") * _NC + lax.axis_index("c")
    base = wid * _BPW

    pltpu.sync_copy(tw.at[pl.ds(base, _BPW)], idx_t)
    pltpu.sync_copy(cw.at[pl.ds(base, _BPW)], idx_c)

    copies = []
    for j in range(_NCH):
        sl = pl.ds(j * _CHUNK, _CHUNK)
        copies.append(pltpu.async_copy(te.at[idx_t.at[sl]], rows_t.at[sl], sem))
        copies.append(pltpu.async_copy(ce.at[idx_c.at[sl]], rows_c.at[sl], sem))
    for cp in copies:
        cp.wait()

    lane = lax.iota(jnp.int32, 16)

    def grp_body(g, tot):
        x = jnp.zeros((16,), jnp.float32)
        for k in range(16):
            r = g * 16 + k
            acc = rows_t[r, pl.ds(0, 16)] * rows_c[r, pl.ds(0, 16)]
            for cb in range(1, _EMB // 16):
                acc = acc + (rows_t[r, pl.ds(cb * 16, 16)]
                             * rows_c[r, pl.ds(cb * 16, 16)])
            x = jnp.where(lane == k, jnp.sum(acc), x)
        u = jnp.exp(-jnp.abs(x))
        s = u / (u + 2.0)
        s2 = s * s
        poly = 1.0 + s2 * (1.0 / 3.0 + s2 * (1.0 / 5.0 + s2 * (
            1.0 / 7.0 + s2 * (1.0 / 9.0 + s2 * (1.0 / 11.0)))))
        return tot + jnp.maximum(-x, 0.0) + 2.0 * (s * poly)

    tot = lax.fori_loop(0, _GROUPS, grp_body, jnp.zeros((16,), jnp.float32))
    accv[...] = tot
    pltpu.sync_copy(accv, out.at[wid])


@jax.jit
def _partials(tw, cw, te, ce):
    mesh = plsc.VectorSubcoreMesh(core_axis_name="c", subcore_axis_name="s")
    run = pl.kernel(
        _body,
        mesh=mesh,
        compiler_params=pltpu.CompilerParams(
            needs_layout_passes=False, use_tc_tiling_on_sc=False),
        out_type=jax.ShapeDtypeStruct((_NW, 16), jnp.float32),
        scratch_types=[
            pltpu.VMEM((_BPW,), jnp.int32),
            pltpu.VMEM((_BPW,), jnp.int32),
            pltpu.VMEM((_BPW, _EMB), jnp.float32),
            pltpu.VMEM((_BPW, _EMB), jnp.float32),
            pltpu.VMEM((16,), jnp.float32),
            pltpu.SemaphoreType.DMA,
        ],
    )
    return run(tw, cw, te, ce)


def kernel(target_word, context_word, target_embeddings, context_embeddings):
    tw = target_word.astype(jnp.int32)
    cw = context_word.astype(jnp.int32)
    part = _partials(tw, cw, target_embeddings, context_embeddings)
    return jnp.sum(part)
